# Initial kernel scaffold; baseline (speedup 1.0000x reference)
import functools

import jax
import jax.numpy as jnp
from jax.experimental import pallas as pl
from jax.experimental.pallas import tpu as pltpu

_TOP_K = 8


def _gate_topk_body(x_ref, wt_ref, b_ref, w_out_ref, i_out_ref):
    g = (
        jnp.dot(
            x_ref[...],
            wt_ref[...],
            preferred_element_type=jnp.float32,
            precision=jax.lax.Precision.HIGHEST,
        )
        + b_ref[...]
    )
    n_experts = g.shape[1]
    lane = jax.lax.broadcasted_iota(jnp.int32, g.shape, 1)
    cur = g
    vals = []
    idxs = []
    for _ in range(_TOP_K):
        m = jnp.max(cur, axis=1, keepdims=True)
        a = jnp.min(jnp.where(cur == m, lane, n_experts), axis=1, keepdims=True)
        vals.append(m)
        idxs.append(a)
        cur = jnp.where(lane == a, -jnp.inf, cur)
    v = jnp.concatenate(vals, axis=1)
    i_out_ref[...] = jnp.concatenate(idxs, axis=1)
    ew = jnp.exp(v - v[:, 0:1])
    w_out_ref[...] = ew / jnp.sum(ew, axis=1, keepdims=True)


@functools.partial(jax.jit, static_argnames=("block_t", "interpret"))
def _gate_topk(x, W, b, block_t=1024, interpret=False):
    n_tokens, hidden = x.shape
    n_experts = W.shape[0]
    wt = W.T
    b2 = b.reshape(1, n_experts)
    grid = (n_tokens // block_t,)
    w_out, i_out = pl.pallas_call(
        _gate_topk_body,
        grid=grid,
        in_specs=[
            pl.BlockSpec((block_t, hidden), lambda i: (i, 0)),
            pl.BlockSpec((hidden, n_experts), lambda i: (0, 0)),
            pl.BlockSpec((1, n_experts), lambda i: (0, 0)),
        ],
        out_specs=[
            pl.BlockSpec((block_t, _TOP_K), lambda i: (i, 0)),
            pl.BlockSpec((block_t, _TOP_K), lambda i: (i, 0)),
        ],
        out_shape=[
            jax.ShapeDtypeStruct((n_tokens, _TOP_K), jnp.float32),
            jax.ShapeDtypeStruct((n_tokens, _TOP_K), jnp.int32),
        ],
        interpret=interpret,
    )(x, wt, b2)
    return w_out, i_out


def kernel(x, W, b):
    return _gate_topk(x, W, b)


# speedup vs baseline: 1.0723x; 1.0723x over previous
import functools

import jax
import jax.numpy as jnp
from jax.experimental import pallas as pl
from jax.experimental.pallas import tpu as pltpu

_TOP_K = 8


def _gate_topk_body(x_ref, wt_ref, b_ref, w_out_ref, i_out_ref):
    g = (
        jnp.dot(
            x_ref[...],
            wt_ref[...],
            preferred_element_type=jnp.float32,
        )
        + b_ref[...]
    )
    n_experts = g.shape[1]
    lane = jax.lax.broadcasted_iota(jnp.int32, g.shape, 1)
    cur = g
    vals = []
    idxs = []
    for _ in range(_TOP_K):
        m = jnp.max(cur, axis=1, keepdims=True)
        a = jnp.min(jnp.where(cur == m, lane, n_experts), axis=1, keepdims=True)
        vals.append(m)
        idxs.append(a)
        cur = jnp.where(lane == a, -jnp.inf, cur)
    v = jnp.concatenate(vals, axis=1)
    i_out_ref[...] = jnp.concatenate(idxs, axis=1)
    ew = jnp.exp(v - v[:, 0:1])
    w_out_ref[...] = ew / jnp.sum(ew, axis=1, keepdims=True)


@functools.partial(jax.jit, static_argnames=("block_t", "interpret"))
def _gate_topk(x, W, b, block_t=1024, interpret=False):
    n_tokens, hidden = x.shape
    n_experts = W.shape[0]
    wt = W.T
    b2 = b.reshape(1, n_experts)
    grid = (n_tokens // block_t,)
    w_out, i_out = pl.pallas_call(
        _gate_topk_body,
        grid=grid,
        in_specs=[
            pl.BlockSpec((block_t, hidden), lambda i: (i, 0)),
            pl.BlockSpec((hidden, n_experts), lambda i: (0, 0)),
            pl.BlockSpec((1, n_experts), lambda i: (0, 0)),
        ],
        out_specs=[
            pl.BlockSpec((block_t, _TOP_K), lambda i: (i, 0)),
            pl.BlockSpec((block_t, _TOP_K), lambda i: (i, 0)),
        ],
        out_shape=[
            jax.ShapeDtypeStruct((n_tokens, _TOP_K), jnp.float32),
            jax.ShapeDtypeStruct((n_tokens, _TOP_K), jnp.int32),
        ],
        interpret=interpret,
    )(x, wt, b2)
    return w_out, i_out


def kernel(x, W, b):
    return _gate_topk(x, W, b)


# speedup vs baseline: 2.0773x; 1.9373x over previous
import functools

import jax
import jax.numpy as jnp
from jax.experimental import pallas as pl
from jax.experimental.pallas import tpu as pltpu

_TOP_K = 8


def _gate_topk_body(x_ref, w_ref, b_ref, w_out_ref, i_out_ref):
    g = (
        jax.lax.dot_general(
            w_ref[...],
            x_ref[...],
            dimension_numbers=(((1,), (1,)), ((), ())),
            preferred_element_type=jnp.float32,
        )
        + b_ref[...]
    )
    n_experts = g.shape[0]
    sub = jax.lax.broadcasted_iota(jnp.int32, g.shape, 0)
    cur = g
    vals = []
    idxs = []
    for _ in range(_TOP_K):
        m = jnp.max(cur, axis=0, keepdims=True)
        a = jnp.min(jnp.where(cur == m, sub, n_experts), axis=0, keepdims=True)
        vals.append(m)
        idxs.append(a)
        cur = jnp.where(sub == a, -jnp.inf, cur)
    v = jnp.concatenate(vals, axis=0)
    idx = jnp.concatenate(idxs, axis=0)
    ew = jnp.exp(v - v[0:1, :])
    wgt = ew / jnp.sum(ew, axis=0, keepdims=True)
    w_out_ref[...] = wgt.T
    i_out_ref[...] = idx.T


@functools.partial(jax.jit, static_argnames=("block_t", "interpret"))
def _gate_topk(x, W, b, block_t=1024, interpret=False):
    n_tokens, hidden = x.shape
    n_experts = W.shape[0]
    b2 = b.reshape(n_experts, 1)
    grid = (n_tokens // block_t,)
    w_out, i_out = pl.pallas_call(
        _gate_topk_body,
        grid=grid,
        in_specs=[
            pl.BlockSpec((block_t, hidden), lambda i: (i, 0)),
            pl.BlockSpec((n_experts, hidden), lambda i: (0, 0)),
            pl.BlockSpec((n_experts, 1), lambda i: (0, 0)),
        ],
        out_specs=[
            pl.BlockSpec((block_t, _TOP_K), lambda i: (i, 0)),
            pl.BlockSpec((block_t, _TOP_K), lambda i: (i, 0)),
        ],
        out_shape=[
            jax.ShapeDtypeStruct((n_tokens, _TOP_K), jnp.float32),
            jax.ShapeDtypeStruct((n_tokens, _TOP_K), jnp.int32),
        ],
        interpret=interpret,
    )(x, W, b2)
    return w_out, i_out


def kernel(x, W, b):
    return _gate_topk(x, W, b)


# speedup vs baseline: 2.3398x; 1.1264x over previous
import functools

import jax
import jax.numpy as jnp
from jax.experimental import pallas as pl
from jax.experimental.pallas import tpu as pltpu

_TOP_K = 8


def _gate_topk_body(x_ref, w_ref, b_ref, w_out_ref, i_out_ref):
    g = (
        jax.lax.dot_general(
            w_ref[...],
            x_ref[...],
            dimension_numbers=(((1,), (1,)), ((), ())),
            preferred_element_type=jnp.float32,
        )
        + b_ref[...]
    )
    n_experts = g.shape[0]
    sub = jax.lax.broadcasted_iota(jnp.int32, g.shape, 0)
    cur = g
    vals = []
    idxs = []
    for _ in range(_TOP_K):
        m = jnp.max(cur, axis=0, keepdims=True)
        a = jnp.min(jnp.where(cur == m, sub, n_experts), axis=0, keepdims=True)
        vals.append(m)
        idxs.append(a)
        cur = jnp.where(sub == a, -jnp.inf, cur)
    v = jnp.concatenate(vals, axis=0)
    idx = jnp.concatenate(idxs, axis=0)
    ew = jnp.exp(v - v[0:1, :])
    wgt = ew / jnp.sum(ew, axis=0, keepdims=True)
    w_out_ref[...] = wgt.T
    i_out_ref[...] = idx.T


@functools.partial(jax.jit, static_argnames=("block_t", "interpret"))
def _gate_topk(x, W, b, block_t=2048, interpret=False):
    n_tokens, hidden = x.shape
    n_experts = W.shape[0]
    b2 = b.reshape(n_experts, 1)
    grid = (n_tokens // block_t,)
    w_out, i_out = pl.pallas_call(
        _gate_topk_body,
        grid=grid,
        in_specs=[
            pl.BlockSpec((block_t, hidden), lambda i: (i, 0)),
            pl.BlockSpec((n_experts, hidden), lambda i: (0, 0)),
            pl.BlockSpec((n_experts, 1), lambda i: (0, 0)),
        ],
        out_specs=[
            pl.BlockSpec((block_t, _TOP_K), lambda i: (i, 0)),
            pl.BlockSpec((block_t, _TOP_K), lambda i: (i, 0)),
        ],
        out_shape=[
            jax.ShapeDtypeStruct((n_tokens, _TOP_K), jnp.float32),
            jax.ShapeDtypeStruct((n_tokens, _TOP_K), jnp.int32),
        ],
        interpret=interpret,
    )(x, W, b2)
    return w_out, i_out


def kernel(x, W, b):
    return _gate_topk(x, W, b)


# speedup vs baseline: 2.4564x; 1.0498x over previous
import functools

import jax
import jax.numpy as jnp
from jax.experimental import pallas as pl
from jax.experimental.pallas import tpu as pltpu

_TOP_K = 8


def _gate_topk_body(x_ref, w_ref, b_ref, w_out_ref, i_out_ref):
    g = (
        jax.lax.dot_general(
            w_ref[...],
            x_ref[...],
            dimension_numbers=(((1,), (1,)), ((), ())),
            preferred_element_type=jnp.float32,
        )
        + b_ref[...]
    )
    n_experts = g.shape[0]
    sub = jax.lax.broadcasted_iota(jnp.int32, g.shape, 0)
    cur = g
    vals = []
    idxs = []
    for _ in range(_TOP_K):
        m = jnp.max(cur, axis=0, keepdims=True)
        a = jnp.min(jnp.where(cur == m, sub, n_experts), axis=0, keepdims=True)
        vals.append(m)
        idxs.append(a)
        cur = jnp.where(sub == a, -jnp.inf, cur)
    v = jnp.concatenate(vals, axis=0)
    idx = jnp.concatenate(idxs, axis=0)
    ew = jnp.exp(v - v[0:1, :])
    wgt = ew / jnp.sum(ew, axis=0, keepdims=True)
    w_out_ref[...] = wgt.T
    i_out_ref[...] = idx.T


@functools.partial(jax.jit, static_argnames=("block_t", "interpret"))
def _gate_topk(x, W, b, block_t=4096, interpret=False):
    n_tokens, hidden = x.shape
    n_experts = W.shape[0]
    b2 = b.reshape(n_experts, 1)
    grid = (n_tokens // block_t,)
    w_out, i_out = pl.pallas_call(
        _gate_topk_body,
        grid=grid,
        in_specs=[
            pl.BlockSpec((block_t, hidden), lambda i: (i, 0)),
            pl.BlockSpec((n_experts, hidden), lambda i: (0, 0)),
            pl.BlockSpec((n_experts, 1), lambda i: (0, 0)),
        ],
        out_specs=[
            pl.BlockSpec((block_t, _TOP_K), lambda i: (i, 0)),
            pl.BlockSpec((block_t, _TOP_K), lambda i: (i, 0)),
        ],
        out_shape=[
            jax.ShapeDtypeStruct((n_tokens, _TOP_K), jnp.float32),
            jax.ShapeDtypeStruct((n_tokens, _TOP_K), jnp.int32),
        ],
        interpret=interpret,
    )(x, W, b2)
    return w_out, i_out


def kernel(x, W, b):
    return _gate_topk(x, W, b)


# speedup vs baseline: 2.5420x; 1.0348x over previous
import functools

import jax
import jax.numpy as jnp
from jax.experimental import pallas as pl
from jax.experimental.pallas import tpu as pltpu

_TOP_K = 8


def _gate_topk_body(x_ref, w_ref, b_ref, w_out_ref, i_out_ref):
    g = (
        jax.lax.dot_general(
            w_ref[...],
            x_ref[...],
            dimension_numbers=(((1,), (1,)), ((), ())),
            preferred_element_type=jnp.float32,
        )
        + b_ref[...]
    )
    sub = jax.lax.broadcasted_iota(jnp.int32, g.shape, 0)
    cur = g
    vals = []
    idxs = []
    for _ in range(_TOP_K):
        v, i = cur, sub
        while v.shape[0] > 1:
            h = v.shape[0] // 2
            c = v[:h] >= v[h:]
            v = jnp.where(c, v[:h], v[h:])
            i = jnp.where(c, i[:h], i[h:])
        vals.append(v)
        idxs.append(i)
        cur = jnp.where(sub == i, -jnp.inf, cur)
    v = jnp.concatenate(vals, axis=0)
    idx = jnp.concatenate(idxs, axis=0)
    ew = jnp.exp(v - v[0:1, :])
    wgt = ew / jnp.sum(ew, axis=0, keepdims=True)
    w_out_ref[...] = wgt.T
    i_out_ref[...] = idx.T


@functools.partial(jax.jit, static_argnames=("block_t", "interpret"))
def _gate_topk(x, W, b, block_t=4096, interpret=False):
    n_tokens, hidden = x.shape
    n_experts = W.shape[0]
    b2 = b.reshape(n_experts, 1)
    grid = (n_tokens // block_t,)
    w_out, i_out = pl.pallas_call(
        _gate_topk_body,
        grid=grid,
        in_specs=[
            pl.BlockSpec((block_t, hidden), lambda i: (i, 0)),
            pl.BlockSpec((n_experts, hidden), lambda i: (0, 0)),
            pl.BlockSpec((n_experts, 1), lambda i: (0, 0)),
        ],
        out_specs=[
            pl.BlockSpec((block_t, _TOP_K), lambda i: (i, 0)),
            pl.BlockSpec((block_t, _TOP_K), lambda i: (i, 0)),
        ],
        out_shape=[
            jax.ShapeDtypeStruct((n_tokens, _TOP_K), jnp.float32),
            jax.ShapeDtypeStruct((n_tokens, _TOP_K), jnp.int32),
        ],
        interpret=interpret,
    )(x, W, b2)
    return w_out, i_out


def kernel(x, W, b):
    return _gate_topk(x, W, b)
